# baseline (speedup 1.0000x reference)
import jax
import jax.numpy as jnp
import numpy as np
from jax.experimental import pallas as pl
from jax.experimental.pallas import tpu as pltpu

_D = 512
_MAX_POS = 4096
_LAYERS = 4
_VOCAB = 256


def _freqs_cis(dim, end, theta=10000.0):
    freqs = 1.0 / (theta ** (jnp.arange(0, dim, 2)[: dim // 2].astype(jnp.float32) / dim))
    t = jnp.arange(end).astype(jnp.float32)
    f = jnp.outer(t, freqs)
    return jnp.concatenate([jnp.cos(f), jnp.sin(f)], axis=-1)


def _gelu(u):
    u = u.astype(jnp.bfloat16)
    c0 = jnp.bfloat16(0.7978845608028654)
    c1 = jnp.bfloat16(0.044715)
    half = jnp.bfloat16(0.5)
    one = jnp.bfloat16(1.0)
    return half * u * (one + jnp.tanh(c0 * (u + c1 * u * u * u)))


def _convnext_kernel(text_ref, emb_ref, freqs_ref, dw_ref, w1_ref, w2_ref,
                     out_ref, pad_ref):
    S = text_ref.shape[1]
    D = _D
    S8 = S // 8
    H = S // 2

    tok = text_ref[0]
    iota = jax.lax.broadcasted_iota(jnp.int32, (S, _VOCAB), 1)
    onehot = (jnp.broadcast_to(tok, (S, _VOCAB)) == iota).astype(jnp.bfloat16)
    h0 = jnp.dot(onehot, emb_ref[...], preferred_element_type=jnp.float32)
    h0 = h0 + freqs_ref[...]
    x = h0

    for p in range(8):
        pad_ref[p, 0:8] = jnp.zeros((8, D), jnp.bfloat16)
        pad_ref[p, 8 + S8:16 + S8] = jnp.zeros((8, D), jnp.bfloat16)

    def write_pad(x, p0):
        for i in range(4):
            pad_ref[p0 + i, 8:8 + S8] = x[i * S8:(i + 1) * S8].astype(jnp.bfloat16)

    def convln(p0, L):
        dw = dw_ref[L]
        blocks = []
        for p in range(p0, p0 + 4):
            y = None
            for k in range(7):
                d = k - 3
                q = (p + d) % 8
                c = (p + d - q) // 8
                t = pad_ref[q, 8 + c:8 + c + S8] * dw[k:k + 1]
                y = t if y is None else y + t
            blocks.append(y)
        y = jnp.concatenate(blocks, axis=0).astype(jnp.float32)
        m = jnp.mean(y, axis=-1, keepdims=True)
        yc = y - m
        v = jnp.mean(yc * yc, axis=-1, keepdims=True)
        return (yc * jax.lax.rsqrt(v + 1e-6)).astype(jnp.bfloat16)


    for L in range(_LAYERS):
        write_pad(x[0:H], 0)
        write_pad(x[H:S], 4)
        y = jnp.concatenate([convln(0, L), convln(4, L)], axis=0)
        u = jnp.dot(y, w1_ref[L], preferred_element_type=jnp.float32)
        g = _gelu(u)
        w = jnp.dot(g, w2_ref[L], preferred_element_type=jnp.float32)
        x = x + w
    out_ref[0] = x


def kernel(text, batch, seq_len, emb, blocks):
    B, S = text.shape
    D = _D
    S8 = S // 8
    text_pm = text.reshape(B, S8, 8).transpose(0, 2, 1).reshape(B, S, 1)
    emb_used = emb[1:_VOCAB + 1].astype(jnp.bfloat16)
    if S <= _MAX_POS:
        freqs = _freqs_cis(D, S)
    else:
        pos = jnp.minimum(jnp.arange(S), _MAX_POS - 1)
        freqs = _freqs_cis(D, _MAX_POS)[pos]
    freqs_pm = freqs.reshape(S8, 8, D).transpose(1, 0, 2).reshape(S, D)
    dws = jnp.stack(
        [jnp.pad(b['dw_w'][:, 0, :].T, ((0, 1), (0, 0))) for b in blocks]
    ).astype(jnp.bfloat16)
    w1s = jnp.stack([b['w1'] for b in blocks]).astype(jnp.bfloat16)
    w2s = jnp.stack([b['w2'] for b in blocks]).astype(jnp.bfloat16)
    out_pm = pl.pallas_call(
        _convnext_kernel,
        grid=(B,),
        in_specs=[
            pl.BlockSpec((1, S, 1), lambda b: (b, 0, 0)),
            pl.BlockSpec((_VOCAB, D), lambda b: (0, 0)),
            pl.BlockSpec((S, D), lambda b: (0, 0)),
            pl.BlockSpec((_LAYERS, 8, D), lambda b: (0, 0, 0)),
            pl.BlockSpec((_LAYERS, D, 2 * D), lambda b: (0, 0, 0)),
            pl.BlockSpec((_LAYERS, 2 * D, D), lambda b: (0, 0, 0)),
        ],
        out_specs=pl.BlockSpec((1, S, D), lambda b: (b, 0, 0)),
        out_shape=jax.ShapeDtypeStruct((B, S, D), jnp.float32),
        scratch_shapes=[pltpu.VMEM((8, S8 + 16, D), jnp.bfloat16)],
        compiler_params=pltpu.CompilerParams(
            dimension_semantics=("arbitrary",),
            vmem_limit_bytes=56 * 1024 * 1024,
        ),
    )(text_pm, emb_used, freqs_pm, dws, w1s, w2s)
    return out_pm.reshape(B, 8, S8, D).transpose(0, 2, 1, 3).reshape(B, S, D)


# speedup vs baseline: 1.0351x; 1.0351x over previous
import jax
import jax.numpy as jnp
import numpy as np
from jax.experimental import pallas as pl
from jax.experimental.pallas import tpu as pltpu

_D = 512
_MAX_POS = 4096
_LAYERS = 4
_VOCAB = 256


def _freqs_cis(dim, end, theta=10000.0):
    freqs = 1.0 / (theta ** (jnp.arange(0, dim, 2)[: dim // 2].astype(jnp.float32) / dim))
    t = jnp.arange(end).astype(jnp.float32)
    f = jnp.outer(t, freqs)
    return jnp.concatenate([jnp.cos(f), jnp.sin(f)], axis=-1)


def _gelu(u):
    u = u.astype(jnp.bfloat16)
    c0 = jnp.bfloat16(0.7978845608028654)
    c1 = jnp.bfloat16(0.044715)
    half = jnp.bfloat16(0.5)
    one = jnp.bfloat16(1.0)
    return half * u * (one + jnp.tanh(c0 * (u + c1 * u * u * u)))


def _convnext_kernel(text_ref, emb_ref, freqs_ref, dw_ref, w1_ref, w2_ref,
                     out_ref, pad_ref):
    S = text_ref.shape[1]
    D = _D
    S8 = S // 8
    H = S // 2

    tok = text_ref[0]
    iota = jax.lax.broadcasted_iota(jnp.int32, (S, _VOCAB), 1)
    onehot = (jnp.broadcast_to(tok, (S, _VOCAB)) == iota).astype(jnp.bfloat16)
    h0 = jnp.dot(onehot, emb_ref[...], preferred_element_type=jnp.float32)
    h0 = h0 + freqs_ref[...]
    xs = [h0[i * (S // 4):(i + 1) * (S // 4)] for i in range(4)]

    for p in range(8):
        pad_ref[p, 0:8] = jnp.zeros((8, D), jnp.bfloat16)
        pad_ref[p, 8 + S8:16 + S8] = jnp.zeros((8, D), jnp.bfloat16)

    def write_pad(x, p0):
        for i in range(2):
            pad_ref[p0 + i, 8:8 + S8] = x[i * S8:(i + 1) * S8].astype(jnp.bfloat16)

    def convln(p0, L):
        dw = dw_ref[L]
        blocks = []
        for p in range(p0, p0 + 2):
            y = None
            for k in range(7):
                d = k - 3
                q = (p + d) % 8
                c = (p + d - q) // 8
                t = pad_ref[q, 8 + c:8 + c + S8] * dw[k:k + 1]
                y = t if y is None else y + t
            blocks.append(y)
        y = jnp.concatenate(blocks, axis=0).astype(jnp.float32)
        m = jnp.mean(y, axis=-1, keepdims=True)
        yc = y - m
        v = jnp.mean(yc * yc, axis=-1, keepdims=True)
        return (yc * jax.lax.rsqrt(v + 1e-6)).astype(jnp.bfloat16)

    for L in range(_LAYERS):
        for i in range(4):
            write_pad(xs[i], 2 * i)
        ys, us, gs, ws = {}, {}, {}, {}
        for t in range(7):
            if 0 <= t - 1 < 4:
                us[t - 1] = jnp.dot(ys[t - 1], w1_ref[L],
                                    preferred_element_type=jnp.float32)
            if 0 <= t - 3 < 4:
                ws[t - 3] = jnp.dot(gs[t - 3], w2_ref[L],
                                    preferred_element_type=jnp.float32)
                xs[t - 3] = xs[t - 3] + ws[t - 3]
            if 0 <= t < 4:
                ys[t] = convln(2 * t, L)
            if 0 <= t - 2 < 4:
                gs[t - 2] = _gelu(us[t - 2])
    for i in range(4):
        out_ref[0, i * (S // 4):(i + 1) * (S // 4)] = xs[i]


def kernel(text, batch, seq_len, emb, blocks):
    B, S = text.shape
    D = _D
    S8 = S // 8
    text_pm = text.reshape(B, S8, 8).transpose(0, 2, 1).reshape(B, S, 1)
    emb_used = emb[1:_VOCAB + 1].astype(jnp.bfloat16)
    if S <= _MAX_POS:
        freqs = _freqs_cis(D, S)
    else:
        pos = jnp.minimum(jnp.arange(S), _MAX_POS - 1)
        freqs = _freqs_cis(D, _MAX_POS)[pos]
    freqs_pm = freqs.reshape(S8, 8, D).transpose(1, 0, 2).reshape(S, D)
    dws = jnp.stack(
        [jnp.pad(b['dw_w'][:, 0, :].T, ((0, 1), (0, 0))) for b in blocks]
    ).astype(jnp.bfloat16)
    w1s = jnp.stack([b['w1'] for b in blocks]).astype(jnp.bfloat16)
    w2s = jnp.stack([b['w2'] for b in blocks]).astype(jnp.bfloat16)
    out_pm = pl.pallas_call(
        _convnext_kernel,
        grid=(B,),
        in_specs=[
            pl.BlockSpec((1, S, 1), lambda b: (b, 0, 0)),
            pl.BlockSpec((_VOCAB, D), lambda b: (0, 0)),
            pl.BlockSpec((S, D), lambda b: (0, 0)),
            pl.BlockSpec((_LAYERS, 8, D), lambda b: (0, 0, 0)),
            pl.BlockSpec((_LAYERS, D, 2 * D), lambda b: (0, 0, 0)),
            pl.BlockSpec((_LAYERS, 2 * D, D), lambda b: (0, 0, 0)),
        ],
        out_specs=pl.BlockSpec((1, S, D), lambda b: (b, 0, 0)),
        out_shape=jax.ShapeDtypeStruct((B, S, D), jnp.float32),
        scratch_shapes=[pltpu.VMEM((8, S8 + 16, D), jnp.bfloat16)],
        compiler_params=pltpu.CompilerParams(
            dimension_semantics=("arbitrary",),
            vmem_limit_bytes=56 * 1024 * 1024,
        ),
    )(text_pm, emb_used, freqs_pm, dws, w1s, w2s)
    return out_pm.reshape(B, 8, S8, D).transpose(0, 2, 1, 3).reshape(B, S, D)
